# baseline (speedup 1.0000x reference)
import functools
import math

import jax
import jax.numpy as jnp
from jax import lax
from jax.experimental import pallas as pl
from jax.experimental.pallas import tpu as pltpu
from jax.experimental.pallas import tpu_sc as plsc

B = 2
L = 100000
C_IN = 128
C_MID = 32
C_OUT = 3
C_OUT_PAD = 8
K = 5
STRIPE = 4096
N_STRIPES = 25
LP = N_STRIPES * STRIPE
CHUNK = 128
NW = 32
N_CHUNKS = LP // CHUNK
CHUNKS_PER_W = N_CHUNKS // NW

OCH = 640
O_GROUPS = OCH // 16
ON_CHUNKS = LP // OCH
OCH_PER_W = ON_CHUNKS // NW
OWIN = OCH + 24

_SC_MESH = dict(core_axis_name="c", subcore_axis_name="s", num_cores=2,
                num_subcores=16)
NEG_INF = float("-inf")


def _sc_gather_rows(idx2, n_vec, x2):
    SPAN = CHUNKS_PER_W * CHUNK

    def body(idx2_hbm, n_hbm, x2_hbm, g_hbm, ib0, ib1, bufa, bufb, nbuf,
             sema, semb, semi):
        wid = lax.axis_index("s") * 2 + lax.axis_index("c")
        pltpu.sync_copy(n_hbm, nbuf)
        n = nbuf[...][0]
        nch = (n + CHUNK - 1) // CHUNK
        ngr = (nch + NW - 1) // NW
        start = wid * ngr * CHUNK

        di0 = pltpu.async_copy(idx2_hbm.at[pl.ds(start, SPAN)], ib0, semi)
        di1 = pltpu.async_copy(idx2_hbm.at[pl.ds(LP + start, SPAN)], ib1, semi)

        bufs = (bufa, bufb)
        sems = (sema, semb)

        def fire(ib, gr, slot):
            return pltpu.async_copy(
                x2_hbm.at[ib.at[pl.ds(gr * CHUNK, CHUNK)]], bufs[slot],
                sems[slot])

        for b, ib, di in ((0, ib0, di0), (1, ib1, di1)):
            di.wait()

            @pl.when(ngr > 0)
            def _():
                fire(ib, 0, 0)

            for gr in range(CHUNKS_PER_W):
                slot = gr % 2

                @pl.when(gr < ngr)
                def _():
                    if gr + 1 < CHUNKS_PER_W:
                        @pl.when(gr + 1 < ngr)
                        def _():
                            fire(ib, gr + 1, 1 - slot)

                    pltpu.make_async_copy(
                        x2_hbm.at[ib.at[pl.ds(gr * CHUNK, CHUNK)]],
                        bufs[slot], sems[slot]).wait()
                    pltpu.sync_copy(
                        bufs[slot],
                        g_hbm.at[b, pl.ds(start + gr * CHUNK, CHUNK)])

    f = pl.kernel(
        body,
        out_type=jax.ShapeDtypeStruct((B, LP, C_IN), jnp.float32),
        mesh=plsc.VectorSubcoreMesh(**_SC_MESH),
        scratch_types=[
            pltpu.VMEM((SPAN,), jnp.int32),
            pltpu.VMEM((SPAN,), jnp.int32),
            pltpu.VMEM((CHUNK, C_IN), jnp.float32),
            pltpu.VMEM((CHUNK, C_IN), jnp.float32),
            pltpu.VMEM((16,), jnp.int32),
            pltpu.SemaphoreType.DMA,
            pltpu.SemaphoreType.DMA,
            pltpu.SemaphoreType.DMA,
        ],
    )
    return f(idx2, n_vec, x2)


def _conv1_body(n_ref, g_ref, w_ref, b1_ref, h_ref):
    s = pl.program_id(1)
    base = s * STRIPE
    n = n_ref[0]

    @pl.when(base < n)
    def _():
        a = g_ref[0]
        in_ranks = base + lax.broadcasted_iota(jnp.int32, (STRIPE, 1), 0)
        a = jnp.where(in_ranks < n, a, 0.0)
        acc = jnp.zeros((C_MID, STRIPE), jnp.float32)
        for k in range(K):
            d = k - (K // 2)
            if d < 0:
                a_d = jnp.concatenate(
                    [jnp.zeros((-d, C_IN), jnp.float32), a[: STRIPE + d]], axis=0)
            elif d > 0:
                a_d = jnp.concatenate(
                    [a[d:], jnp.zeros((d, C_IN), jnp.float32)], axis=0)
            else:
                a_d = a
            acc = acc + lax.dot_general(
                w_ref[k], a_d, (((1,), (1,)), ((), ())),
                preferred_element_type=jnp.float32)
        acc = acc + b1_ref[...]
        y = 0.5 * acc * (1.0 + lax.erf(acc * (1.0 / math.sqrt(2.0))))
        out_ranks = base + lax.broadcasted_iota(jnp.int32, (C_MID, STRIPE), 1)
        h_ref[0] = jnp.where(out_ranks < n, y, 0.0)

    @pl.when(base >= n)
    def _():
        h_ref[0] = jnp.zeros((C_MID, STRIPE), jnp.float32)


def _conv1(g, w1f, b1, n_arr):
    return pl.pallas_call(
        _conv1_body,
        grid=(B, N_STRIPES),
        in_specs=[
            pl.BlockSpec(memory_space=pltpu.SMEM),
            pl.BlockSpec((1, STRIPE, C_IN), lambda b, s: (b, s, 0)),
            pl.BlockSpec((K, C_MID, C_IN), lambda b, s: (0, 0, 0)),
            pl.BlockSpec((C_MID, 1), lambda b, s: (0, 0)),
        ],
        out_specs=pl.BlockSpec((1, C_MID, STRIPE), lambda b, s: (b, 0, s)),
        out_shape=jax.ShapeDtypeStruct((B, C_MID, LP), jnp.float32),
    )(n_arr, g, w1f, b1)


def _conv2_body(n_ref, hp_ref, hc_ref, hn_ref, b2_ref, w_ref, y_ref):
    s = pl.program_id(1)
    base = s * STRIPE
    n = n_ref[0]

    @pl.when(base < n)
    def _():
        cur = hc_ref[0]
        left = jnp.where(s > 0, hp_ref[0][:, STRIPE - 2:], 0.0)
        right = jnp.where(s < N_STRIPES - 1, hn_ref[0][:, :2], 0.0)
        hx = jnp.concatenate([left, cur, right], axis=1)
        acc = jnp.zeros((C_OUT_PAD, STRIPE), jnp.float32)
        for k in range(K):
            acc = acc + lax.dot_general(
                w_ref[k], hx[:, k:k + STRIPE], (((1,), (0,)), ((), ())),
                preferred_element_type=jnp.float32)
        acc = acc + b2_ref[...]
        ranks = base + lax.broadcasted_iota(jnp.int32, (C_OUT_PAD, STRIPE), 1)
        y_ref[0] = jnp.where(ranks < n, acc, NEG_INF)

    @pl.when(base >= n)
    def _():
        y_ref[0] = jnp.full((C_OUT_PAD, STRIPE), NEG_INF, jnp.float32)


def _conv2(h, w2f, b2p, n_arr):
    return pl.pallas_call(
        _conv2_body,
        grid=(B, N_STRIPES),
        in_specs=[
            pl.BlockSpec(memory_space=pltpu.SMEM),
            pl.BlockSpec((1, C_MID, STRIPE),
                         lambda b, s: (b, 0, jnp.maximum(s - 1, 0))),
            pl.BlockSpec((1, C_MID, STRIPE), lambda b, s: (b, 0, s)),
            pl.BlockSpec((1, C_MID, STRIPE),
                         lambda b, s: (b, 0, jnp.minimum(s + 1, N_STRIPES - 1))),
            pl.BlockSpec((C_OUT_PAD, 1), lambda b, s: (0, 0)),
            pl.BlockSpec((K, C_OUT_PAD, C_MID), lambda b, s: (0, 0, 0)),
        ],
        out_specs=pl.BlockSpec((1, C_OUT_PAD, STRIPE), lambda b, s: (b, 0, s)),
        out_shape=jax.ShapeDtypeStruct((B, C_OUT_PAD, LP), jnp.float32),
    )(n_arr, h, h, h, b2p, w2f)


def _sc_expand_out(maskp, exp, y1d):
    def body(mask_hbm, ex_hbm, y_hbm, out_hbm, mbuf, exbuf, ybufs, obufs, sem):
        wid = lax.axis_index("s") * 2 + lax.axis_index("c")
        neg_inf_v = jnp.full((16,), NEG_INF, jnp.float32)

        def step(i, carry):
            base = (wid + NW * i) * OCH
            pltpu.sync_copy(mask_hbm.at[pl.ds(base, OCH)], mbuf)
            pltpu.sync_copy(ex_hbm.at[pl.ds(base, 16)], exbuf)
            r0 = exbuf[...][0]
            a0 = (r0 // 8) * 8
            off0 = r0 - a0
            for b in range(B):
                for c in range(C_OUT):
                    pltpu.sync_copy(
                        y_hbm.at[pl.ds((b * C_OUT_PAD + c) * LP + a0, OWIN)],
                        ybufs[b * C_OUT + c])
            off = off0
            for g in range(O_GROUPS):
                mi = mbuf[pl.ds(g * 16, 16)]
                m = mi != 0
                ci = plsc.cumsum(mi)
                idxv = ci - mi + off
                for b in range(B):
                    for c in range(C_OUT):
                        v = plsc.load_gather(ybufs[b * C_OUT + c], [idxv])
                        obufs[b * C_OUT + c][pl.ds(g * 16, 16)] = (
                            jnp.where(m, v, neg_inf_v))
                off = off + ci[15]
            for b in range(B):
                for c in range(C_OUT):
                    pltpu.sync_copy(
                        obufs[b * C_OUT + c],
                        out_hbm.at[pl.ds((b * C_OUT + c) * LP + base, OCH)])
            return carry

        lax.fori_loop(0, OCH_PER_W, step, 0)

    f = pl.kernel(
        body,
        out_type=jax.ShapeDtypeStruct((B * C_OUT * LP,), jnp.float32),
        mesh=plsc.VectorSubcoreMesh(**_SC_MESH),
        compiler_params=pltpu.CompilerParams(needs_layout_passes=False),
        scratch_types=[
            pltpu.VMEM((OCH,), jnp.int32),
            pltpu.VMEM((16,), jnp.int32),
            [pltpu.VMEM((OWIN,), jnp.float32) for _ in range(B * C_OUT)],
            [pltpu.VMEM((OCH,), jnp.float32) for _ in range(B * C_OUT)],
            pltpu.SemaphoreType.DMA,
        ],
    )
    return f(maskp, exp, y1d)


def kernel(x, mask, w1, b1, w2, b2):
    active = (mask != 0).astype(jnp.int32)
    cum = jnp.cumsum(active)
    n = cum[-1]
    n_arr = jnp.broadcast_to(n, (1,))
    n_vec = jnp.broadcast_to(n, (16,))

    idx = jnp.nonzero(active, size=L, fill_value=L - 1)[0].astype(jnp.int32)
    idxp = jnp.pad(idx, (0, LP - L), constant_values=L - 1)
    idx2 = jnp.concatenate([idxp, idxp + L])

    ex = cum - active
    exp = jnp.pad(ex, (0, LP + 16 - L), mode="edge").astype(jnp.int32)
    maskp = jnp.pad(active, (0, LP - L))

    x2 = x.reshape(B * L, C_IN)
    g = _sc_gather_rows(idx2, n_vec, x2)

    w1f = jnp.transpose(w1, (2, 0, 1))
    h = _conv1(g, w1f, b1.reshape(C_MID, 1), n_arr)

    w2f = jnp.transpose(w2, (2, 0, 1))
    w2f = jnp.pad(w2f, ((0, 0), (0, C_OUT_PAD - C_OUT), (0, 0)))
    b2p = jnp.pad(b2, (0, C_OUT_PAD - C_OUT)).reshape(C_OUT_PAD, 1)
    y3 = _conv2(h, w2f, b2p, n_arr)

    out_p = _sc_expand_out(maskp, exp, y3.reshape(B * C_OUT_PAD * LP))
    out_p = out_p.reshape(B, C_OUT, LP)
    return jnp.transpose(out_p[:, :, :L], (0, 2, 1))


# speedup vs baseline: 1.9013x; 1.9013x over previous
import functools
import math

import jax
import jax.numpy as jnp
from jax import lax
from jax.experimental import pallas as pl
from jax.experimental.pallas import tpu as pltpu
from jax.experimental.pallas import tpu_sc as plsc

B = 2
L = 100000
C_IN = 128
C_MID = 32
C_OUT = 3
C_OUT_PAD = 8
K = 5
STRIPE = 4096
N_STRIPES = 25
LP = N_STRIPES * STRIPE
CHUNK = 128
NW = 32
N_CHUNKS = LP // CHUNK
CHUNKS_PER_W = N_CHUNKS // NW

OCH = 640
O_GROUPS = OCH // 16
ON_CHUNKS = LP // OCH
OCH_PER_W = ON_CHUNKS // NW
OWIN = OCH + 24

_SC_MESH = dict(core_axis_name="c", subcore_axis_name="s", num_cores=2,
                num_subcores=16)
NEG_INF = float("-inf")


def _sc_gather_rows(idx2, n_vec, x2):
    def body(idx2_hbm, n_hbm, x2_hbm, g_hbm, ib0, ib1, bufa, bufb, nbuf,
             sema, semb):
        wid = lax.axis_index("s") * 2 + lax.axis_index("c")
        pltpu.sync_copy(n_hbm, nbuf)
        n = nbuf[...][0]

        ibufs = (ib0, ib1)
        bufs = (bufa, bufb)
        sems = (sema, semb)

        def chunk_base(i):
            return (wid + NW * i) * CHUNK

        def stage(boff, i, slot):
            pltpu.sync_copy(
                idx2_hbm.at[pl.ds(boff + chunk_base(i), CHUNK)], ibufs[slot])
            pltpu.async_copy(x2_hbm.at[ibufs[slot]], bufs[slot], sems[slot])

        for b in range(B):
            boff = b * LP

            @pl.when(chunk_base(0) < n)
            def _():
                stage(boff, 0, 0)

            for i in range(CHUNKS_PER_W):
                slot = i % 2
                base = chunk_base(i)

                @pl.when(base < n)
                def _():
                    if i + 1 < CHUNKS_PER_W:
                        @pl.when(chunk_base(i + 1) < n)
                        def _():
                            stage(boff, i + 1, 1 - slot)
                    pltpu.make_async_copy(
                        x2_hbm.at[ibufs[slot]], bufs[slot], sems[slot]).wait()
                    pltpu.sync_copy(bufs[slot],
                                    g_hbm.at[b, pl.ds(base, CHUNK)])

    f = pl.kernel(
        body,
        out_type=jax.ShapeDtypeStruct((B, LP, C_IN), jnp.float32),
        mesh=plsc.VectorSubcoreMesh(**_SC_MESH),
        scratch_types=[
            pltpu.VMEM((CHUNK,), jnp.int32),
            pltpu.VMEM((CHUNK,), jnp.int32),
            pltpu.VMEM((CHUNK, C_IN), jnp.float32),
            pltpu.VMEM((CHUNK, C_IN), jnp.float32),
            pltpu.VMEM((16,), jnp.int32),
            pltpu.SemaphoreType.DMA,
            pltpu.SemaphoreType.DMA,
        ],
    )
    return f(idx2, n_vec, x2)


def _conv1_body(n_ref, g_ref, w_ref, b1_ref, h_ref):
    s = pl.program_id(1)
    base = s * STRIPE
    n = n_ref[0]

    @pl.when(base < n)
    def _():
        a = g_ref[0]
        in_ranks = base + lax.broadcasted_iota(jnp.int32, (STRIPE, 1), 0)
        a = jnp.where(in_ranks < n, a, 0.0)
        acc = jnp.zeros((C_MID, STRIPE), jnp.float32)
        for k in range(K):
            d = k - (K // 2)
            if d < 0:
                a_d = jnp.concatenate(
                    [jnp.zeros((-d, C_IN), jnp.float32), a[: STRIPE + d]], axis=0)
            elif d > 0:
                a_d = jnp.concatenate(
                    [a[d:], jnp.zeros((d, C_IN), jnp.float32)], axis=0)
            else:
                a_d = a
            acc = acc + lax.dot_general(
                w_ref[k], a_d, (((1,), (1,)), ((), ())),
                preferred_element_type=jnp.float32)
        acc = acc + b1_ref[...]
        y = 0.5 * acc * (1.0 + lax.erf(acc * (1.0 / math.sqrt(2.0))))
        out_ranks = base + lax.broadcasted_iota(jnp.int32, (C_MID, STRIPE), 1)
        h_ref[0] = jnp.where(out_ranks < n, y, 0.0)

    @pl.when(base >= n)
    def _():
        h_ref[0] = jnp.zeros((C_MID, STRIPE), jnp.float32)


def _conv1(g, w1f, b1, n_arr):
    return pl.pallas_call(
        _conv1_body,
        grid=(B, N_STRIPES),
        in_specs=[
            pl.BlockSpec(memory_space=pltpu.SMEM),
            pl.BlockSpec((1, STRIPE, C_IN), lambda b, s: (b, s, 0)),
            pl.BlockSpec((K, C_MID, C_IN), lambda b, s: (0, 0, 0)),
            pl.BlockSpec((C_MID, 1), lambda b, s: (0, 0)),
        ],
        out_specs=pl.BlockSpec((1, C_MID, STRIPE), lambda b, s: (b, 0, s)),
        out_shape=jax.ShapeDtypeStruct((B, C_MID, LP), jnp.float32),
    )(n_arr, g, w1f, b1)


def _conv2_body(n_ref, hp_ref, hc_ref, hn_ref, b2_ref, w_ref, y_ref):
    s = pl.program_id(1)
    base = s * STRIPE
    n = n_ref[0]

    @pl.when(base < n)
    def _():
        cur = hc_ref[0]
        left = jnp.where(s > 0, hp_ref[0][:, STRIPE - 2:], 0.0)
        right = jnp.where(s < N_STRIPES - 1, hn_ref[0][:, :2], 0.0)
        hx = jnp.concatenate([left, cur, right], axis=1)
        acc = jnp.zeros((C_OUT_PAD, STRIPE), jnp.float32)
        for k in range(K):
            acc = acc + lax.dot_general(
                w_ref[k], hx[:, k:k + STRIPE], (((1,), (0,)), ((), ())),
                preferred_element_type=jnp.float32)
        acc = acc + b2_ref[...]
        ranks = base + lax.broadcasted_iota(jnp.int32, (C_OUT_PAD, STRIPE), 1)
        y_ref[0] = jnp.where(ranks < n, acc, NEG_INF)

    @pl.when(base >= n)
    def _():
        y_ref[0] = jnp.full((C_OUT_PAD, STRIPE), NEG_INF, jnp.float32)


def _conv2(h, w2f, b2p, n_arr):
    return pl.pallas_call(
        _conv2_body,
        grid=(B, N_STRIPES),
        in_specs=[
            pl.BlockSpec(memory_space=pltpu.SMEM),
            pl.BlockSpec((1, C_MID, STRIPE),
                         lambda b, s: (b, 0, jnp.maximum(s - 1, 0))),
            pl.BlockSpec((1, C_MID, STRIPE), lambda b, s: (b, 0, s)),
            pl.BlockSpec((1, C_MID, STRIPE),
                         lambda b, s: (b, 0, jnp.minimum(s + 1, N_STRIPES - 1))),
            pl.BlockSpec((C_OUT_PAD, 1), lambda b, s: (0, 0)),
            pl.BlockSpec((K, C_OUT_PAD, C_MID), lambda b, s: (0, 0, 0)),
        ],
        out_specs=pl.BlockSpec((1, C_OUT_PAD, STRIPE), lambda b, s: (b, 0, s)),
        out_shape=jax.ShapeDtypeStruct((B, C_OUT_PAD, LP), jnp.float32),
    )(n_arr, h, h, h, b2p, w2f)


def _sc_expand_out(maskp, exp, y1d):
    def body(mask_hbm, ex_hbm, y_hbm, out_hbm, mbuf, exbuf, ybufs, obufs, sem):
        wid = lax.axis_index("s") * 2 + lax.axis_index("c")
        neg_inf_v = jnp.full((16,), NEG_INF, jnp.float32)

        def step(i, carry):
            base = (wid + NW * i) * OCH
            pltpu.sync_copy(mask_hbm.at[pl.ds(base, OCH)], mbuf)
            pltpu.sync_copy(ex_hbm.at[pl.ds(base, 16)], exbuf)
            r0 = exbuf[...][0]
            a0 = (r0 // 8) * 8
            off0 = r0 - a0
            for b in range(B):
                for c in range(C_OUT):
                    pltpu.sync_copy(
                        y_hbm.at[pl.ds((b * C_OUT_PAD + c) * LP + a0, OWIN)],
                        ybufs[b * C_OUT + c])
            off = off0
            for g in range(O_GROUPS):
                mi = mbuf[pl.ds(g * 16, 16)]
                m = mi != 0
                ci = plsc.cumsum(mi)
                idxv = ci - mi + off
                for b in range(B):
                    for c in range(C_OUT):
                        v = plsc.load_gather(ybufs[b * C_OUT + c], [idxv])
                        obufs[b * C_OUT + c][pl.ds(g * 16, 16)] = (
                            jnp.where(m, v, neg_inf_v))
                off = off + ci[15]
            for b in range(B):
                for c in range(C_OUT):
                    pltpu.sync_copy(
                        obufs[b * C_OUT + c],
                        out_hbm.at[pl.ds((b * C_OUT + c) * LP + base, OCH)])
            return carry

        lax.fori_loop(0, OCH_PER_W, step, 0)

    f = pl.kernel(
        body,
        out_type=jax.ShapeDtypeStruct((B * C_OUT * LP,), jnp.float32),
        mesh=plsc.VectorSubcoreMesh(**_SC_MESH),
        compiler_params=pltpu.CompilerParams(needs_layout_passes=False),
        scratch_types=[
            pltpu.VMEM((OCH,), jnp.int32),
            pltpu.VMEM((16,), jnp.int32),
            [pltpu.VMEM((OWIN,), jnp.float32) for _ in range(B * C_OUT)],
            [pltpu.VMEM((OCH,), jnp.float32) for _ in range(B * C_OUT)],
            pltpu.SemaphoreType.DMA,
        ],
    )
    return f(maskp, exp, y1d)


def kernel(x, mask, w1, b1, w2, b2):
    active = (mask != 0).astype(jnp.int32)
    cum = jnp.cumsum(active)
    n = cum[-1]
    n_arr = jnp.broadcast_to(n, (1,))
    n_vec = jnp.broadcast_to(n, (16,))

    idx = jnp.nonzero(active, size=L, fill_value=L - 1)[0].astype(jnp.int32)
    idxp = jnp.pad(idx, (0, LP - L), constant_values=L - 1)
    idx2 = jnp.concatenate([idxp, idxp + L])

    ex = cum - active
    exp = jnp.pad(ex, (0, LP + 16 - L), mode="edge").astype(jnp.int32)
    maskp = jnp.pad(active, (0, LP - L))

    x2 = x.reshape(B * L, C_IN)
    g = _sc_gather_rows(idx2, n_vec, x2)

    w1f = jnp.transpose(w1, (2, 0, 1))
    h = _conv1(g, w1f, b1.reshape(C_MID, 1), n_arr)

    w2f = jnp.transpose(w2, (2, 0, 1))
    w2f = jnp.pad(w2f, ((0, 0), (0, C_OUT_PAD - C_OUT), (0, 0)))
    b2p = jnp.pad(b2, (0, C_OUT_PAD - C_OUT)).reshape(C_OUT_PAD, 1)
    y3 = _conv2(h, w2f, b2p, n_arr)

    out_p = _sc_expand_out(maskp, exp, y3.reshape(B * C_OUT_PAD * LP))
    out_p = out_p.reshape(B, C_OUT, LP)
    return jnp.transpose(out_p[:, :, :L], (0, 2, 1))


# speedup vs baseline: 2.2068x; 1.1607x over previous
import functools
import math

import jax
import jax.numpy as jnp
from jax import lax
from jax.experimental import pallas as pl
from jax.experimental.pallas import tpu as pltpu
from jax.experimental.pallas import tpu_sc as plsc

B = 2
L = 100000
C_IN = 128
C_MID = 32
C_OUT = 3
C_OUT_PAD = 8
K = 5
STRIPE = 4096
N_STRIPES = 25
LP = N_STRIPES * STRIPE
CHUNK = 128
NW = 32
N_CHUNKS = LP // CHUNK
CHUNKS_PER_W = N_CHUNKS // NW

OCH = 640
O_GROUPS = OCH // 16
ON_CHUNKS = LP // OCH
OCH_PER_W = ON_CHUNKS // NW
OWIN = OCH + 24

_SC_MESH = dict(core_axis_name="c", subcore_axis_name="s", num_cores=2,
                num_subcores=16)
NEG_INF = float("-inf")


def _sc_gather_rows(idx2, n_vec, x2):
    def body(idx2_hbm, n_hbm, x2_hbm, g_hbm, ib0, ib1, bufa, bufb, nbuf,
             sema, semb):
        wid = lax.axis_index("s") * 2 + lax.axis_index("c")
        pltpu.sync_copy(n_hbm, nbuf)
        n = nbuf[...][0]

        ibufs = (ib0, ib1)
        bufs = (bufa, bufb)
        sems = (sema, semb)

        def chunk_base(i):
            return (wid + NW * i) * CHUNK

        def stage(boff, i, slot):
            pltpu.sync_copy(
                idx2_hbm.at[pl.ds(boff + chunk_base(i), CHUNK)], ibufs[slot])
            pltpu.async_copy(x2_hbm.at[ibufs[slot]], bufs[slot], sems[slot])

        for b in range(B):
            boff = b * LP

            @pl.when(chunk_base(0) < n)
            def _():
                stage(boff, 0, 0)

            for i in range(CHUNKS_PER_W):
                slot = i % 2
                base = chunk_base(i)

                @pl.when(base < n)
                def _():
                    if i + 1 < CHUNKS_PER_W:
                        @pl.when(chunk_base(i + 1) < n)
                        def _():
                            stage(boff, i + 1, 1 - slot)
                    pltpu.make_async_copy(
                        x2_hbm.at[ibufs[slot]], bufs[slot], sems[slot]).wait()
                    pltpu.sync_copy(bufs[slot],
                                    g_hbm.at[b, pl.ds(base, CHUNK)])

    f = pl.kernel(
        body,
        out_type=jax.ShapeDtypeStruct((B, LP, C_IN), jnp.float32),
        mesh=plsc.VectorSubcoreMesh(**_SC_MESH),
        scratch_types=[
            pltpu.VMEM((CHUNK,), jnp.int32),
            pltpu.VMEM((CHUNK,), jnp.int32),
            pltpu.VMEM((CHUNK, C_IN), jnp.float32),
            pltpu.VMEM((CHUNK, C_IN), jnp.float32),
            pltpu.VMEM((16,), jnp.int32),
            pltpu.SemaphoreType.DMA,
            pltpu.SemaphoreType.DMA,
        ],
    )
    return f(idx2, n_vec, x2)


def _convs_body(n_ref, g_ref, w1_ref, b1_ref, w2_ref, b2_ref, y_ref,
                h2_ref, tail_ref):
    s = pl.program_id(1)
    n = n_ref[0]

    @pl.when(s < N_STRIPES)
    def _():
        base = s * STRIPE

        @pl.when(base < n)
        def _():
            a = g_ref[0]
            in_ranks = base + lax.broadcasted_iota(jnp.int32, (STRIPE, 1), 0)
            a = jnp.where(in_ranks < n, a, 0.0)
            acc = jnp.zeros((C_MID, STRIPE), jnp.float32)
            for k in range(K):
                d = k - (K // 2)
                if d < 0:
                    a_d = jnp.concatenate(
                        [jnp.zeros((-d, C_IN), jnp.float32), a[: STRIPE + d]],
                        axis=0)
                elif d > 0:
                    a_d = jnp.concatenate(
                        [a[d:], jnp.zeros((d, C_IN), jnp.float32)], axis=0)
                else:
                    a_d = a
                acc = acc + lax.dot_general(
                    w1_ref[k], a_d, (((1,), (1,)), ((), ())),
                    preferred_element_type=jnp.float32)
            acc = acc + b1_ref[...]
            y = 0.5 * acc * (1.0 + lax.erf(acc * (1.0 / math.sqrt(2.0))))
            out_ranks = base + lax.broadcasted_iota(jnp.int32, (C_MID, STRIPE), 1)
            h2_ref[s % 2] = jnp.where(out_ranks < n, y, 0.0)

        @pl.when(base >= n)
        def _():
            h2_ref[s % 2] = jnp.zeros((C_MID, STRIPE), jnp.float32)

    @pl.when(s >= 1)
    def _():
        sp = s - 1
        base_p = sp * STRIPE

        @pl.when(base_p < n)
        def _():
            cur = h2_ref[(s - 1) % 2]
            left = jnp.where(sp > 0, tail_ref[:, C_IN - 2:], 0.0)
            right = jnp.where(sp < N_STRIPES - 1, h2_ref[s % 2][:, :2], 0.0)
            hx = jnp.concatenate([left, cur, right], axis=1)
            acc = jnp.zeros((C_OUT_PAD, STRIPE), jnp.float32)
            for k in range(K):
                acc = acc + lax.dot_general(
                    w2_ref[k], hx[:, k:k + STRIPE], (((1,), (0,)), ((), ())),
                    preferred_element_type=jnp.float32)
            acc = acc + b2_ref[...]
            ranks = base_p + lax.broadcasted_iota(
                jnp.int32, (C_OUT_PAD, STRIPE), 1)
            y_ref[0] = jnp.where(ranks < n, acc, NEG_INF)

        @pl.when(base_p >= n)
        def _():
            y_ref[0] = jnp.full((C_OUT_PAD, STRIPE), NEG_INF, jnp.float32)

        tail_ref[...] = h2_ref[(s - 1) % 2][:, STRIPE - C_IN:]


def _convs(g, w1f, b1, w2f, b2p, n_arr):
    return pl.pallas_call(
        _convs_body,
        grid=(B, N_STRIPES + 1),
        in_specs=[
            pl.BlockSpec(memory_space=pltpu.SMEM),
            pl.BlockSpec((1, STRIPE, C_IN),
                         lambda b, s: (b, jnp.minimum(s, N_STRIPES - 1), 0)),
            pl.BlockSpec((K, C_MID, C_IN), lambda b, s: (0, 0, 0)),
            pl.BlockSpec((C_MID, 1), lambda b, s: (0, 0)),
            pl.BlockSpec((K, C_OUT_PAD, C_MID), lambda b, s: (0, 0, 0)),
            pl.BlockSpec((C_OUT_PAD, 1), lambda b, s: (0, 0)),
        ],
        out_specs=pl.BlockSpec((1, C_OUT_PAD, STRIPE),
                               lambda b, s: (b, 0, jnp.maximum(s - 1, 0))),
        out_shape=jax.ShapeDtypeStruct((B, C_OUT_PAD, LP), jnp.float32),
        scratch_shapes=[
            pltpu.VMEM((2, C_MID, STRIPE), jnp.float32),
            pltpu.VMEM((C_MID, C_IN), jnp.float32),
        ],
    )(n_arr, g, w1f, b1, w2f, b2p)


def _sc_expand_out(maskp, exp, y1d):
    def body(mask_hbm, ex_hbm, y_hbm, out_hbm, mbuf, exbuf, ybufs, obufs, sem):
        wid = lax.axis_index("s") * 2 + lax.axis_index("c")
        neg_inf_v = jnp.full((16,), NEG_INF, jnp.float32)

        def step(i, carry):
            base = (wid + NW * i) * OCH
            pltpu.sync_copy(mask_hbm.at[pl.ds(base, OCH)], mbuf)
            pltpu.sync_copy(ex_hbm.at[pl.ds(base, 16)], exbuf)
            r0 = exbuf[...][0]
            a0 = (r0 // 8) * 8
            off0 = r0 - a0
            for b in range(B):
                for c in range(C_OUT):
                    pltpu.sync_copy(
                        y_hbm.at[pl.ds((b * C_OUT_PAD + c) * LP + a0, OWIN)],
                        ybufs[b * C_OUT + c])
            off = off0
            for g in range(O_GROUPS):
                mi = mbuf[pl.ds(g * 16, 16)]
                m = mi != 0
                ci = plsc.cumsum(mi)
                idxv = ci - mi + off
                for b in range(B):
                    for c in range(C_OUT):
                        v = plsc.load_gather(ybufs[b * C_OUT + c], [idxv])
                        obufs[b * C_OUT + c][pl.ds(g * 16, 16)] = (
                            jnp.where(m, v, neg_inf_v))
                off = off + ci[15]
            for b in range(B):
                for c in range(C_OUT):
                    pltpu.sync_copy(
                        obufs[b * C_OUT + c],
                        out_hbm.at[pl.ds((b * C_OUT + c) * LP + base, OCH)])
            return carry

        lax.fori_loop(0, OCH_PER_W, step, 0)

    f = pl.kernel(
        body,
        out_type=jax.ShapeDtypeStruct((B * C_OUT * LP,), jnp.float32),
        mesh=plsc.VectorSubcoreMesh(**_SC_MESH),
        compiler_params=pltpu.CompilerParams(needs_layout_passes=False),
        scratch_types=[
            pltpu.VMEM((OCH,), jnp.int32),
            pltpu.VMEM((16,), jnp.int32),
            [pltpu.VMEM((OWIN,), jnp.float32) for _ in range(B * C_OUT)],
            [pltpu.VMEM((OCH,), jnp.float32) for _ in range(B * C_OUT)],
            pltpu.SemaphoreType.DMA,
        ],
    )
    return f(maskp, exp, y1d)


def kernel(x, mask, w1, b1, w2, b2):
    active = (mask != 0).astype(jnp.int32)
    cum = jnp.cumsum(active)
    n = cum[-1]
    n_arr = jnp.broadcast_to(n, (1,))
    n_vec = jnp.broadcast_to(n, (16,))

    idx = jnp.nonzero(active, size=L, fill_value=L - 1)[0].astype(jnp.int32)
    idxp = jnp.pad(idx, (0, LP - L), constant_values=L - 1)
    idx2 = jnp.concatenate([idxp, idxp + L])

    ex = cum - active
    exp = jnp.pad(ex, (0, LP + 16 - L), mode="edge").astype(jnp.int32)
    maskp = jnp.pad(active, (0, LP - L))

    x2 = x.reshape(B * L, C_IN)
    g = _sc_gather_rows(idx2, n_vec, x2)

    w1f = jnp.transpose(w1, (2, 0, 1))
    w2f = jnp.transpose(w2, (2, 0, 1))
    w2f = jnp.pad(w2f, ((0, 0), (0, C_OUT_PAD - C_OUT), (0, 0)))
    b2p = jnp.pad(b2, (0, C_OUT_PAD - C_OUT)).reshape(C_OUT_PAD, 1)
    y3 = _convs(g, w1f, b1.reshape(C_MID, 1), w2f, b2p, n_arr)

    out_p = _sc_expand_out(maskp, exp, y3.reshape(B * C_OUT_PAD * LP))
    out_p = out_p.reshape(B, C_OUT, LP)
    return jnp.transpose(out_p[:, :, :L], (0, 2, 1))


# speedup vs baseline: 2.3186x; 1.0507x over previous
import functools
import math

import jax
import jax.numpy as jnp
from jax import lax
from jax.experimental import pallas as pl
from jax.experimental.pallas import tpu as pltpu
from jax.experimental.pallas import tpu_sc as plsc

B = 2
L = 100000
C_IN = 128
C_MID = 32
C_OUT = 3
C_OUT_PAD = 8
K = 5
STRIPE = 4096
N_STRIPES = 25
LP = N_STRIPES * STRIPE
CHUNK = 128
NW = 32
N_CHUNKS = LP // CHUNK
CHUNKS_PER_W = N_CHUNKS // NW

OCH = 640
O_GROUPS = OCH // 16
ON_CHUNKS = LP // OCH
OCH_PER_W = ON_CHUNKS // NW
OWIN = OCH + 24

_SC_MESH = dict(core_axis_name="c", subcore_axis_name="s", num_cores=2,
                num_subcores=16)
NEG_INF = float("-inf")


def _sc_gather_rows(idx2, n_vec, x2):
    def body(idx2_hbm, n_hbm, x2_hbm, g_hbm, ib0, ib1, bufa, bufb, nbuf,
             sema, semb):
        wid = lax.axis_index("s") * 2 + lax.axis_index("c")
        pltpu.sync_copy(n_hbm, nbuf)
        n = nbuf[...][0]

        ibufs = (ib0, ib1)
        bufs = (bufa, bufb)
        sems = (sema, semb)

        def chunk_base(i):
            return (wid + NW * i) * CHUNK

        def stage(boff, i, slot):
            pltpu.sync_copy(
                idx2_hbm.at[pl.ds(boff + chunk_base(i), CHUNK)], ibufs[slot])
            pltpu.async_copy(x2_hbm.at[ibufs[slot]], bufs[slot], sems[slot])

        for b in range(B):
            boff = b * LP

            @pl.when(chunk_base(0) < n)
            def _():
                stage(boff, 0, 0)

            for i in range(CHUNKS_PER_W):
                slot = i % 2
                base = chunk_base(i)

                @pl.when(base < n)
                def _():
                    if i + 1 < CHUNKS_PER_W:
                        @pl.when(chunk_base(i + 1) < n)
                        def _():
                            stage(boff, i + 1, 1 - slot)
                    pltpu.make_async_copy(
                        x2_hbm.at[ibufs[slot]], bufs[slot], sems[slot]).wait()
                    pltpu.sync_copy(bufs[slot],
                                    g_hbm.at[b, pl.ds(base, CHUNK)])

    f = pl.kernel(
        body,
        out_type=jax.ShapeDtypeStruct((B, LP, C_IN), jnp.float32),
        mesh=plsc.VectorSubcoreMesh(**_SC_MESH),
        scratch_types=[
            pltpu.VMEM((CHUNK,), jnp.int32),
            pltpu.VMEM((CHUNK,), jnp.int32),
            pltpu.VMEM((CHUNK, C_IN), jnp.float32),
            pltpu.VMEM((CHUNK, C_IN), jnp.float32),
            pltpu.VMEM((16,), jnp.int32),
            pltpu.SemaphoreType.DMA,
            pltpu.SemaphoreType.DMA,
        ],
    )
    return f(idx2, n_vec, x2)


def _convs_body(n_ref, g_ref, w1_ref, b1_ref, w2_ref, b2_ref, y_ref,
                h2_ref, tail_ref):
    s = pl.program_id(1)
    n = n_ref[0]

    @pl.when(s < N_STRIPES)
    def _():
        base = s * STRIPE

        @pl.when(base < n)
        def _():
            a = g_ref[0]
            in_ranks = base + lax.broadcasted_iota(jnp.int32, (STRIPE, 1), 0)
            a = jnp.where(in_ranks < n, a, 0.0)
            acc = jnp.zeros((C_MID, STRIPE), jnp.float32)
            for k in range(K):
                d = k - (K // 2)
                if d < 0:
                    a_d = jnp.concatenate(
                        [jnp.zeros((-d, C_IN), jnp.float32), a[: STRIPE + d]],
                        axis=0)
                elif d > 0:
                    a_d = jnp.concatenate(
                        [a[d:], jnp.zeros((d, C_IN), jnp.float32)], axis=0)
                else:
                    a_d = a
                acc = acc + lax.dot_general(
                    w1_ref[k], a_d, (((1,), (1,)), ((), ())),
                    preferred_element_type=jnp.float32)
            acc = acc + b1_ref[...]
            y = 0.5 * acc * (1.0 + lax.erf(acc * (1.0 / math.sqrt(2.0))))
            out_ranks = base + lax.broadcasted_iota(jnp.int32, (C_MID, STRIPE), 1)
            h2_ref[s % 2] = jnp.where(out_ranks < n, y, 0.0)

        @pl.when(base >= n)
        def _():
            h2_ref[s % 2] = jnp.zeros((C_MID, STRIPE), jnp.float32)

    @pl.when(s >= 1)
    def _():
        sp = s - 1
        base_p = sp * STRIPE

        @pl.when(base_p < n)
        def _():
            cur = h2_ref[(s - 1) % 2]
            left = jnp.where(sp > 0, tail_ref[:, C_IN - 2:], 0.0)
            right = jnp.where(sp < N_STRIPES - 1, h2_ref[s % 2][:, :2], 0.0)
            hx = jnp.concatenate([left, cur, right], axis=1)
            acc = jnp.zeros((C_OUT_PAD, STRIPE), jnp.float32)
            for k in range(K):
                acc = acc + lax.dot_general(
                    w2_ref[k], hx[:, k:k + STRIPE], (((1,), (0,)), ((), ())),
                    preferred_element_type=jnp.float32)
            acc = acc + b2_ref[...]
            ranks = base_p + lax.broadcasted_iota(
                jnp.int32, (C_OUT_PAD, STRIPE), 1)
            y_ref[0] = jnp.where(ranks < n, acc, NEG_INF)

        @pl.when(base_p >= n)
        def _():
            y_ref[0] = jnp.full((C_OUT_PAD, STRIPE), NEG_INF, jnp.float32)

        tail_ref[...] = h2_ref[(s - 1) % 2][:, STRIPE - C_IN:]


def _g_index(b, s, n_ref):
    last_active = jnp.maximum((n_ref[0] + STRIPE - 1) // STRIPE - 1, 0)
    return (b, jnp.minimum(jnp.minimum(s, N_STRIPES - 1), last_active), 0)


def _convs(g, w1f, b1, w2f, b2p, n_arr):
    return pl.pallas_call(
        _convs_body,
        grid_spec=pltpu.PrefetchScalarGridSpec(
            num_scalar_prefetch=1,
            grid=(B, N_STRIPES + 1),
            in_specs=[
                pl.BlockSpec((1, STRIPE, C_IN), _g_index),
                pl.BlockSpec((K, C_MID, C_IN), lambda b, s, n_ref: (0, 0, 0)),
                pl.BlockSpec((C_MID, 1), lambda b, s, n_ref: (0, 0)),
                pl.BlockSpec((K, C_OUT_PAD, C_MID),
                             lambda b, s, n_ref: (0, 0, 0)),
                pl.BlockSpec((C_OUT_PAD, 1), lambda b, s, n_ref: (0, 0)),
            ],
            out_specs=pl.BlockSpec(
                (1, C_OUT_PAD, STRIPE),
                lambda b, s, n_ref: (b, 0, jnp.maximum(s - 1, 0))),
            scratch_shapes=[
                pltpu.VMEM((2, C_MID, STRIPE), jnp.float32),
                pltpu.VMEM((C_MID, C_IN), jnp.float32),
            ],
        ),
        out_shape=jax.ShapeDtypeStruct((B, C_OUT_PAD, LP), jnp.float32),
    )(n_arr, g, w1f, b1, w2f, b2p)


def _sc_expand_out(maskp, exp, y1d):
    def body(mask_hbm, ex_hbm, y_hbm, out_hbm, mbuf, exbuf, ybufs, obufs, sem):
        wid = lax.axis_index("s") * 2 + lax.axis_index("c")
        neg_inf_v = jnp.full((16,), NEG_INF, jnp.float32)

        def step(i, carry):
            base = (wid + NW * i) * OCH
            pltpu.sync_copy(mask_hbm.at[pl.ds(base, OCH)], mbuf)
            pltpu.sync_copy(ex_hbm.at[pl.ds(base, 16)], exbuf)
            r0 = exbuf[...][0]
            a0 = (r0 // 8) * 8
            off0 = r0 - a0
            for b in range(B):
                for c in range(C_OUT):
                    pltpu.sync_copy(
                        y_hbm.at[pl.ds((b * C_OUT_PAD + c) * LP + a0, OWIN)],
                        ybufs[b * C_OUT + c])
            off = off0
            for g in range(O_GROUPS):
                mi = mbuf[pl.ds(g * 16, 16)]
                m = mi != 0
                ci = plsc.cumsum(mi)
                idxv = ci - mi + off
                for b in range(B):
                    for c in range(C_OUT):
                        v = plsc.load_gather(ybufs[b * C_OUT + c], [idxv])
                        obufs[b * C_OUT + c][pl.ds(g * 16, 16)] = (
                            jnp.where(m, v, neg_inf_v))
                off = off + ci[15]
            for b in range(B):
                for c in range(C_OUT):
                    pltpu.sync_copy(
                        obufs[b * C_OUT + c],
                        out_hbm.at[pl.ds((b * C_OUT + c) * LP + base, OCH)])
            return carry

        lax.fori_loop(0, OCH_PER_W, step, 0)

    f = pl.kernel(
        body,
        out_type=jax.ShapeDtypeStruct((B * C_OUT * LP,), jnp.float32),
        mesh=plsc.VectorSubcoreMesh(**_SC_MESH),
        compiler_params=pltpu.CompilerParams(needs_layout_passes=False),
        scratch_types=[
            pltpu.VMEM((OCH,), jnp.int32),
            pltpu.VMEM((16,), jnp.int32),
            [pltpu.VMEM((OWIN,), jnp.float32) for _ in range(B * C_OUT)],
            [pltpu.VMEM((OCH,), jnp.float32) for _ in range(B * C_OUT)],
            pltpu.SemaphoreType.DMA,
        ],
    )
    return f(maskp, exp, y1d)


def kernel(x, mask, w1, b1, w2, b2):
    active = (mask != 0).astype(jnp.int32)
    cum = jnp.cumsum(active)
    n = cum[-1]
    n_arr = jnp.broadcast_to(n, (1,))
    n_vec = jnp.broadcast_to(n, (16,))

    idx = jnp.nonzero(active, size=L, fill_value=L - 1)[0].astype(jnp.int32)
    idxp = jnp.pad(idx, (0, LP - L), constant_values=L - 1)
    idx2 = jnp.concatenate([idxp, idxp + L])

    ex = cum - active
    exp = jnp.pad(ex, (0, LP + 16 - L), mode="edge").astype(jnp.int32)
    maskp = jnp.pad(active, (0, LP - L))

    x2 = x.reshape(B * L, C_IN)
    g = _sc_gather_rows(idx2, n_vec, x2)

    w1f = jnp.transpose(w1, (2, 0, 1))
    w2f = jnp.transpose(w2, (2, 0, 1))
    w2f = jnp.pad(w2f, ((0, 0), (0, C_OUT_PAD - C_OUT), (0, 0)))
    b2p = jnp.pad(b2, (0, C_OUT_PAD - C_OUT)).reshape(C_OUT_PAD, 1)
    y3 = _convs(g, w1f, b1.reshape(C_MID, 1), w2f, b2p, n_arr)

    out_p = _sc_expand_out(maskp, exp, y3.reshape(B * C_OUT_PAD * LP))
    out_p = out_p.reshape(B, C_OUT, LP)
    return jnp.transpose(out_p[:, :, :L], (0, 2, 1))


# speedup vs baseline: 2.4770x; 1.0683x over previous
import functools
import math

import jax
import jax.numpy as jnp
from jax import lax
from jax.experimental import pallas as pl
from jax.experimental.pallas import tpu as pltpu
from jax.experimental.pallas import tpu_sc as plsc

B = 2
L = 100000
C_IN = 128
C_MID = 32
C_OUT = 3
C_OUT_PAD = 8
K = 5
STRIPE = 4096
N_STRIPES = 25
LP = N_STRIPES * STRIPE
CHUNK = 128
NW = 32
N_CHUNKS = LP // CHUNK
CHUNKS_PER_W = N_CHUNKS // NW

OCH = 640
O_GROUPS = OCH // 16
ON_CHUNKS = LP // OCH
OCH_PER_W = ON_CHUNKS // NW
OWIN = OCH + 24

_SC_MESH = dict(core_axis_name="c", subcore_axis_name="s", num_cores=2,
                num_subcores=16)
NEG_INF = float("-inf")


def _sc_gather_rows(idx2, n_vec, x2):
    def body(idx2_hbm, n_hbm, x2_hbm, g_hbm, ib0, ib1, bufa, bufb, nbuf,
             sema, semb):
        wid = lax.axis_index("s") * 2 + lax.axis_index("c")
        pltpu.sync_copy(n_hbm, nbuf)
        n = nbuf[...][0]

        ibufs = (ib0, ib1)
        bufs = (bufa, bufb)
        sems = (sema, semb)

        def chunk_base(i):
            return (wid + NW * i) * CHUNK

        def stage(boff, i, slot):
            pltpu.sync_copy(
                idx2_hbm.at[pl.ds(boff + chunk_base(i), CHUNK)], ibufs[slot])
            pltpu.async_copy(x2_hbm.at[ibufs[slot]], bufs[slot], sems[slot])

        for b in range(B):
            boff = b * LP

            @pl.when(chunk_base(0) < n)
            def _():
                stage(boff, 0, 0)

            for i in range(CHUNKS_PER_W):
                slot = i % 2
                base = chunk_base(i)

                @pl.when(base < n)
                def _():
                    if i + 1 < CHUNKS_PER_W:
                        @pl.when(chunk_base(i + 1) < n)
                        def _():
                            stage(boff, i + 1, 1 - slot)
                    pltpu.make_async_copy(
                        x2_hbm.at[ibufs[slot]], bufs[slot], sems[slot]).wait()
                    pltpu.sync_copy(bufs[slot],
                                    g_hbm.at[b, pl.ds(base, CHUNK)])

    f = pl.kernel(
        body,
        out_type=jax.ShapeDtypeStruct((B, LP, C_IN), jnp.float32),
        mesh=plsc.VectorSubcoreMesh(**_SC_MESH),
        scratch_types=[
            pltpu.VMEM((CHUNK,), jnp.int32),
            pltpu.VMEM((CHUNK,), jnp.int32),
            pltpu.VMEM((CHUNK, C_IN), jnp.float32),
            pltpu.VMEM((CHUNK, C_IN), jnp.float32),
            pltpu.VMEM((16,), jnp.int32),
            pltpu.SemaphoreType.DMA,
            pltpu.SemaphoreType.DMA,
        ],
    )
    return f(idx2, n_vec, x2)


def _convs_body(n_ref, g_ref, w1_ref, b1_ref, w2_ref, b2_ref, y_ref,
                h2_ref, tail_ref):
    s = pl.program_id(1)
    n = n_ref[0]

    @pl.when(s < N_STRIPES)
    def _():
        base = s * STRIPE

        @pl.when(base < n)
        def _():
            a = g_ref[0]
            in_ranks = base + lax.broadcasted_iota(jnp.int32, (STRIPE, 1), 0)
            a = jnp.where(in_ranks < n, a, 0.0)
            acc = jnp.zeros((C_MID, STRIPE), jnp.float32)
            for k in range(K):
                d = k - (K // 2)
                if d < 0:
                    a_d = jnp.concatenate(
                        [jnp.zeros((-d, C_IN), jnp.float32), a[: STRIPE + d]],
                        axis=0)
                elif d > 0:
                    a_d = jnp.concatenate(
                        [a[d:], jnp.zeros((d, C_IN), jnp.float32)], axis=0)
                else:
                    a_d = a
                acc = acc + lax.dot_general(
                    w1_ref[k], a_d, (((1,), (1,)), ((), ())),
                    preferred_element_type=jnp.float32)
            acc = acc + b1_ref[...]
            y = 0.5 * acc * (1.0 + lax.erf(acc * (1.0 / math.sqrt(2.0))))
            out_ranks = base + lax.broadcasted_iota(jnp.int32, (C_MID, STRIPE), 1)
            h2_ref[s % 2] = jnp.where(out_ranks < n, y, 0.0)

        @pl.when(base >= n)
        def _():
            h2_ref[s % 2] = jnp.zeros((C_MID, STRIPE), jnp.float32)

    @pl.when(s >= 1)
    def _():
        sp = s - 1
        base_p = sp * STRIPE

        @pl.when(base_p < n)
        def _():
            cur = h2_ref[(s - 1) % 2]
            left = jnp.where(sp > 0, tail_ref[:, C_IN - 2:], 0.0)
            right = jnp.where(sp < N_STRIPES - 1, h2_ref[s % 2][:, :2], 0.0)
            hx = jnp.concatenate([left, cur, right], axis=1)
            acc = jnp.zeros((C_OUT_PAD, STRIPE), jnp.float32)
            for k in range(K):
                acc = acc + lax.dot_general(
                    w2_ref[k], hx[:, k:k + STRIPE], (((1,), (0,)), ((), ())),
                    preferred_element_type=jnp.float32)
            acc = acc + b2_ref[...]
            ranks = base_p + lax.broadcasted_iota(
                jnp.int32, (C_OUT_PAD, STRIPE), 1)
            y_ref[0] = jnp.where(ranks < n, acc, NEG_INF)

        @pl.when(base_p >= n)
        def _():
            y_ref[0] = jnp.full((C_OUT_PAD, STRIPE), NEG_INF, jnp.float32)

        tail_ref[...] = h2_ref[(s - 1) % 2][:, STRIPE - C_IN:]


def _g_index(b, s, n_ref):
    last_active = jnp.maximum((n_ref[0] + STRIPE - 1) // STRIPE - 1, 0)
    return (b, jnp.minimum(jnp.minimum(s, N_STRIPES - 1), last_active), 0)


def _convs(g, w1f, b1, w2f, b2p, n_arr):
    return pl.pallas_call(
        _convs_body,
        grid_spec=pltpu.PrefetchScalarGridSpec(
            num_scalar_prefetch=1,
            grid=(B, N_STRIPES + 1),
            in_specs=[
                pl.BlockSpec((1, STRIPE, C_IN), _g_index),
                pl.BlockSpec((K, C_MID, C_IN), lambda b, s, n_ref: (0, 0, 0)),
                pl.BlockSpec((C_MID, 1), lambda b, s, n_ref: (0, 0)),
                pl.BlockSpec((K, C_OUT_PAD, C_MID),
                             lambda b, s, n_ref: (0, 0, 0)),
                pl.BlockSpec((C_OUT_PAD, 1), lambda b, s, n_ref: (0, 0)),
            ],
            out_specs=pl.BlockSpec(
                (1, C_OUT_PAD, STRIPE),
                lambda b, s, n_ref: (b, 0, jnp.maximum(s - 1, 0))),
            scratch_shapes=[
                pltpu.VMEM((2, C_MID, STRIPE), jnp.float32),
                pltpu.VMEM((C_MID, C_IN), jnp.float32),
            ],
        ),
        out_shape=jax.ShapeDtypeStruct((B, C_OUT_PAD, LP), jnp.float32),
    )(n_arr, g, w1f, b1, w2f, b2p)


def _sc_expand_out(maskp, exp, y1d):
    def body(mask_hbm, ex_hbm, y_hbm, out_hbm, mbuf, exbuf, ybufs, obufs, sem):
        wid = lax.axis_index("s") * 2 + lax.axis_index("c")
        neg_inf_v = jnp.full((16,), NEG_INF, jnp.float32)

        def step(i, carry):
            base = (wid + NW * i) * OCH
            dm = pltpu.async_copy(mask_hbm.at[pl.ds(base, OCH)], mbuf, sem)
            de = pltpu.async_copy(ex_hbm.at[pl.ds(base, 16)], exbuf, sem)
            dm.wait()
            de.wait()
            r0 = exbuf[...][0]
            a0 = (r0 // 8) * 8
            off0 = r0 - a0
            dy = []
            for b in range(B):
                for c in range(C_OUT):
                    dy.append(pltpu.async_copy(
                        y_hbm.at[pl.ds((b * C_OUT_PAD + c) * LP + a0, OWIN)],
                        ybufs[b * C_OUT + c], sem))
            for d in dy:
                d.wait()
            off = off0
            for g in range(O_GROUPS):
                mi = mbuf[pl.ds(g * 16, 16)]
                m = mi != 0
                ci = plsc.cumsum(mi)
                idxv = ci - mi + off
                for b in range(B):
                    for c in range(C_OUT):
                        v = plsc.load_gather(ybufs[b * C_OUT + c], [idxv])
                        obufs[b * C_OUT + c][pl.ds(g * 16, 16)] = (
                            jnp.where(m, v, neg_inf_v))
                off = off + ci[15]
            dw = []
            for b in range(B):
                for c in range(C_OUT):
                    dw.append(pltpu.async_copy(
                        obufs[b * C_OUT + c],
                        out_hbm.at[pl.ds((b * C_OUT + c) * LP + base, OCH)],
                        sem))
            for d in dw:
                d.wait()
            return carry

        lax.fori_loop(0, OCH_PER_W, step, 0)

    f = pl.kernel(
        body,
        out_type=jax.ShapeDtypeStruct((B * C_OUT * LP,), jnp.float32),
        mesh=plsc.VectorSubcoreMesh(**_SC_MESH),
        compiler_params=pltpu.CompilerParams(needs_layout_passes=False),
        scratch_types=[
            pltpu.VMEM((OCH,), jnp.int32),
            pltpu.VMEM((16,), jnp.int32),
            [pltpu.VMEM((OWIN,), jnp.float32) for _ in range(B * C_OUT)],
            [pltpu.VMEM((OCH,), jnp.float32) for _ in range(B * C_OUT)],
            pltpu.SemaphoreType.DMA,
        ],
    )
    return f(maskp, exp, y1d)


def kernel(x, mask, w1, b1, w2, b2):
    active = (mask != 0).astype(jnp.int32)
    cum = jnp.cumsum(active)
    n = cum[-1]
    n_arr = jnp.broadcast_to(n, (1,))
    n_vec = jnp.broadcast_to(n, (16,))

    idx = jnp.nonzero(active, size=L, fill_value=L - 1)[0].astype(jnp.int32)
    idxp = jnp.pad(idx, (0, LP - L), constant_values=L - 1)
    idx2 = jnp.concatenate([idxp, idxp + L])

    ex = cum - active
    exp = jnp.pad(ex, (0, LP + 16 - L), mode="edge").astype(jnp.int32)
    maskp = jnp.pad(active, (0, LP - L))

    x2 = x.reshape(B * L, C_IN)
    g = _sc_gather_rows(idx2, n_vec, x2)

    w1f = jnp.transpose(w1, (2, 0, 1))
    w2f = jnp.transpose(w2, (2, 0, 1))
    w2f = jnp.pad(w2f, ((0, 0), (0, C_OUT_PAD - C_OUT), (0, 0)))
    b2p = jnp.pad(b2, (0, C_OUT_PAD - C_OUT)).reshape(C_OUT_PAD, 1)
    y3 = _convs(g, w1f, b1.reshape(C_MID, 1), w2f, b2p, n_arr)

    out_p = _sc_expand_out(maskp, exp, y3.reshape(B * C_OUT_PAD * LP))
    out_p = out_p.reshape(B, C_OUT, LP)
    return jnp.transpose(out_p[:, :, :L], (0, 2, 1))


# speedup vs baseline: 2.4948x; 1.0072x over previous
import math

import jax
import jax.numpy as jnp
from jax import lax
from jax.experimental import pallas as pl
from jax.experimental.pallas import tpu as pltpu
from jax.experimental.pallas import tpu_sc as plsc

B = 2
L = 100000
C_IN = 128
C_MID = 32
C_OUT = 3
C_OUT_PAD = 8
K = 5
STRIPE = 4096
N_STRIPES = 25
LP = N_STRIPES * STRIPE
CHUNK = 128
NW = 32
N_CHUNKS = LP // CHUNK
CHUNKS_PER_W = N_CHUNKS // NW

OCH = 640
O_GROUPS = OCH // 16
ON_CHUNKS = LP // OCH
OCH_PER_W = ON_CHUNKS // NW
OWIN = OCH + 24

_SC_MESH = dict(core_axis_name="c", subcore_axis_name="s", num_cores=2,
                num_subcores=16)
NEG_INF = float("-inf")


def _sc_gather_rows(idx2, n_vec, x2):
    def body(idx2_hbm, n_hbm, x2_hbm, g_hbm, ib0, ib1, bufa, bufb, nbuf,
             sema, semb):
        wid = lax.axis_index("s") * 2 + lax.axis_index("c")
        pltpu.sync_copy(n_hbm, nbuf)
        n = nbuf[...][0]

        ibufs = (ib0, ib1)
        bufs = (bufa, bufb)
        sems = (sema, semb)

        def chunk_base(i):
            return (wid + NW * i) * CHUNK

        def stage(boff, i, slot):
            pltpu.sync_copy(
                idx2_hbm.at[pl.ds(boff + chunk_base(i), CHUNK)], ibufs[slot])
            pltpu.async_copy(x2_hbm.at[ibufs[slot]], bufs[slot], sems[slot])

        for b in range(B):
            boff = b * LP

            @pl.when(chunk_base(0) < n)
            def _():
                stage(boff, 0, 0)

            for i in range(CHUNKS_PER_W):
                slot = i % 2
                base = chunk_base(i)

                @pl.when(base < n)
                def _():
                    if i + 1 < CHUNKS_PER_W:
                        @pl.when(chunk_base(i + 1) < n)
                        def _():
                            stage(boff, i + 1, 1 - slot)
                    pltpu.make_async_copy(
                        x2_hbm.at[ibufs[slot]], bufs[slot], sems[slot]).wait()
                    pltpu.sync_copy(bufs[slot],
                                    g_hbm.at[b, pl.ds(base, CHUNK)])

    f = pl.kernel(
        body,
        out_type=jax.ShapeDtypeStruct((B, LP, C_IN), jnp.float32),
        mesh=plsc.VectorSubcoreMesh(**_SC_MESH),
        scratch_types=[
            pltpu.VMEM((CHUNK,), jnp.int32),
            pltpu.VMEM((CHUNK,), jnp.int32),
            pltpu.VMEM((CHUNK, C_IN), jnp.float32),
            pltpu.VMEM((CHUNK, C_IN), jnp.float32),
            pltpu.VMEM((16,), jnp.int32),
            pltpu.SemaphoreType.DMA,
            pltpu.SemaphoreType.DMA,
        ],
    )
    return f(idx2, n_vec, x2)


def _convs_body(n_ref, g_ref, w1_ref, b1_ref, w2_ref, b2_ref, y_ref,
                h2_ref, tail_ref):
    s = pl.program_id(1)
    n = n_ref[0]

    @pl.when(s < N_STRIPES)
    def _():
        base = s * STRIPE

        @pl.when(base < n)
        def _():
            a = g_ref[0]
            in_ranks = base + lax.broadcasted_iota(jnp.int32, (STRIPE, 1), 0)
            a = jnp.where(in_ranks < n, a, 0.0)
            acc = jnp.zeros((C_MID, STRIPE), jnp.float32)
            for k in range(K):
                d = k - (K // 2)
                if d < 0:
                    a_d = jnp.concatenate(
                        [jnp.zeros((-d, C_IN), jnp.float32), a[: STRIPE + d]],
                        axis=0)
                elif d > 0:
                    a_d = jnp.concatenate(
                        [a[d:], jnp.zeros((d, C_IN), jnp.float32)], axis=0)
                else:
                    a_d = a
                acc = acc + lax.dot_general(
                    w1_ref[k], a_d, (((1,), (1,)), ((), ())),
                    preferred_element_type=jnp.float32)
            acc = acc + b1_ref[...]
            y = 0.5 * acc * (1.0 + lax.erf(acc * (1.0 / math.sqrt(2.0))))
            out_ranks = base + lax.broadcasted_iota(jnp.int32, (C_MID, STRIPE), 1)
            h2_ref[s % 2] = jnp.where(out_ranks < n, y, 0.0)

        @pl.when(base >= n)
        def _():
            h2_ref[s % 2] = jnp.zeros((C_MID, STRIPE), jnp.float32)

    @pl.when(s >= 1)
    def _():
        sp = s - 1
        base_p = sp * STRIPE

        @pl.when(base_p < n)
        def _():
            cur = h2_ref[(s - 1) % 2]
            left = jnp.where(sp > 0, tail_ref[:, C_IN - 2:], 0.0)
            right = jnp.where(sp < N_STRIPES - 1, h2_ref[s % 2][:, :2], 0.0)
            hx = jnp.concatenate([left, cur, right], axis=1)
            acc = jnp.zeros((C_OUT_PAD, STRIPE), jnp.float32)
            for k in range(K):
                acc = acc + lax.dot_general(
                    w2_ref[k], hx[:, k:k + STRIPE], (((1,), (0,)), ((), ())),
                    preferred_element_type=jnp.float32)
            acc = acc + b2_ref[...]
            ranks = base_p + lax.broadcasted_iota(
                jnp.int32, (C_OUT_PAD, STRIPE), 1)
            y_ref[0] = jnp.where(ranks < n, acc, NEG_INF)

        @pl.when(base_p >= n)
        def _():
            y_ref[0] = jnp.full((C_OUT_PAD, STRIPE), NEG_INF, jnp.float32)

        tail_ref[...] = h2_ref[(s - 1) % 2][:, STRIPE - C_IN:]


def _g_index(b, s, n_ref):
    last_active = jnp.maximum((n_ref[0] + STRIPE - 1) // STRIPE - 1, 0)
    return (b, jnp.minimum(jnp.minimum(s, N_STRIPES - 1), last_active), 0)


def _convs(g, w1f, b1, w2f, b2p, n_arr):
    return pl.pallas_call(
        _convs_body,
        grid_spec=pltpu.PrefetchScalarGridSpec(
            num_scalar_prefetch=1,
            grid=(B, N_STRIPES + 1),
            in_specs=[
                pl.BlockSpec((1, STRIPE, C_IN), _g_index),
                pl.BlockSpec((K, C_MID, C_IN), lambda b, s, n_ref: (0, 0, 0)),
                pl.BlockSpec((C_MID, 1), lambda b, s, n_ref: (0, 0)),
                pl.BlockSpec((K, C_OUT_PAD, C_MID),
                             lambda b, s, n_ref: (0, 0, 0)),
                pl.BlockSpec((C_OUT_PAD, 1), lambda b, s, n_ref: (0, 0)),
            ],
            out_specs=pl.BlockSpec(
                (1, C_OUT_PAD, STRIPE),
                lambda b, s, n_ref: (b, 0, jnp.maximum(s - 1, 0))),
            scratch_shapes=[
                pltpu.VMEM((2, C_MID, STRIPE), jnp.float32),
                pltpu.VMEM((C_MID, C_IN), jnp.float32),
            ],
        ),
        out_shape=jax.ShapeDtypeStruct((B, C_OUT_PAD, LP), jnp.float32),
    )(n_arr, g, w1f, b1, w2f, b2p)


def _sc_expand_out(maskp, exp, y1d):
    def body(mask_hbm, ex_hbm, y_hbm, out_hbm, mbuf, exbuf, ybufs, obufs, sem):
        wid = lax.axis_index("s") * 2 + lax.axis_index("c")
        neg_inf_v = jnp.full((16,), NEG_INF, jnp.float32)

        def step(i, carry):
            base = (wid + NW * i) * OCH
            dm = pltpu.async_copy(mask_hbm.at[pl.ds(base, OCH)], mbuf, sem)
            de = pltpu.async_copy(ex_hbm.at[pl.ds(base, 16)], exbuf, sem)
            dm.wait()
            de.wait()
            r0 = exbuf[...][0]
            a0 = (r0 // 8) * 8
            off0 = r0 - a0
            dy = []
            for b in range(B):
                for c in range(C_OUT):
                    dy.append(pltpu.async_copy(
                        y_hbm.at[pl.ds((b * C_OUT_PAD + c) * LP + a0, OWIN)],
                        ybufs[b * C_OUT + c], sem))
            for d in dy:
                d.wait()
            off = off0
            for g in range(O_GROUPS):
                mi = mbuf[pl.ds(g * 16, 16)]
                m = mi != 0
                ci = plsc.cumsum(mi)
                idxv = ci - mi + off
                for b in range(B):
                    for c in range(C_OUT):
                        v = plsc.load_gather(ybufs[b * C_OUT + c], [idxv])
                        obufs[b * C_OUT + c][pl.ds(g * 16, 16)] = (
                            jnp.where(m, v, neg_inf_v))
                off = off + ci[15]
            dw = []
            for b in range(B):
                for c in range(C_OUT):
                    dw.append(pltpu.async_copy(
                        obufs[b * C_OUT + c],
                        out_hbm.at[pl.ds((b * C_OUT + c) * LP + base, OCH)],
                        sem))
            for d in dw:
                d.wait()
            return carry

        lax.fori_loop(0, OCH_PER_W, step, 0)

    f = pl.kernel(
        body,
        out_type=jax.ShapeDtypeStruct((B * C_OUT * LP,), jnp.float32),
        mesh=plsc.VectorSubcoreMesh(**_SC_MESH),
        compiler_params=pltpu.CompilerParams(needs_layout_passes=False),
        scratch_types=[
            pltpu.VMEM((OCH,), jnp.int32),
            pltpu.VMEM((16,), jnp.int32),
            [pltpu.VMEM((OWIN,), jnp.float32) for _ in range(B * C_OUT)],
            [pltpu.VMEM((OCH,), jnp.float32) for _ in range(B * C_OUT)],
            pltpu.SemaphoreType.DMA,
        ],
    )
    return f(maskp, exp, y1d)


def kernel(x, mask, w1, b1, w2, b2):
    active = (mask != 0).astype(jnp.int32)
    cum = jnp.cumsum(active)
    n = cum[-1]
    n_arr = jnp.broadcast_to(n, (1,))
    n_vec = jnp.broadcast_to(n, (16,))

    idx = jnp.nonzero(active, size=L, fill_value=L - 1)[0].astype(jnp.int32)
    idxp = jnp.pad(idx, (0, LP - L), constant_values=L - 1)
    idx2 = jnp.concatenate([idxp, idxp + L])

    ex = cum - active
    exp = jnp.pad(ex, (0, LP + 16 - L), mode="edge").astype(jnp.int32)
    maskp = jnp.pad(active, (0, LP - L))

    x2 = x.reshape(B * L, C_IN)
    g = _sc_gather_rows(idx2, n_vec, x2)

    w1f = jnp.transpose(w1, (2, 0, 1))
    w2f = jnp.transpose(w2, (2, 0, 1))
    w2f = jnp.pad(w2f, ((0, 0), (0, C_OUT_PAD - C_OUT), (0, 0)))
    b2p = jnp.pad(b2, (0, C_OUT_PAD - C_OUT)).reshape(C_OUT_PAD, 1)
    y3 = _convs(g, w1f, b1.reshape(C_MID, 1), w2f, b2p, n_arr)

    out_p = _sc_expand_out(maskp, exp, y3.reshape(B * C_OUT_PAD * LP))
    out_p = out_p.reshape(B, C_OUT, LP)
    return jnp.transpose(out_p[:, :, :L], (0, 2, 1))
